# Initial kernel scaffold; baseline (speedup 1.0000x reference)
import jax
import jax.numpy as jnp
from jax.experimental import pallas as pl


def kernel(data, W_enc, b_enc, W_msg, b_msg, W_upd, b_upd, W_mst, b_mst, W_pred, b_pred):
    raise NotImplementedError("write your pallas kernel here")


# speedup vs baseline: 4.6747x; 4.6747x over previous
import functools

import jax
import jax.numpy as jnp
from jax import lax
from jax.experimental import pallas as pl
from jax.experimental.pallas import tpu as pltpu
from jax.experimental.pallas import tpu_sc as plsc

_N = 11
_LD = 16

_R_WENC = 0
_R_WMSG = 17
_R_WUPD = 50
_R_BENC = 98
_R_BMSG = 99
_R_BUPD = 100
_R_WMST = 101
_R_WPRED = 103
_R_SCAL = 107
_R_DATA = 108
_ROWS = 120


def _sqrt16(a):
    i = lax.bitcast_convert_type(a, jnp.int32)
    i = 0x5F3759DF - lax.shift_right_logical(i, 1)
    y = lax.bitcast_convert_type(i, jnp.float32)
    for _ in range(3):
        y = y * (1.5 - 0.5 * a * y * y)
    return a * y


def _lanesum(v):
    lane = lax.iota(jnp.int32, _LD)
    for sh in (1, 2, 4, 8):
        idx = jnp.bitwise_xor(lane, jnp.int32(sh))
        v = v + v.at[idx].get(mode="promise_in_bounds")
    return v


def _bcast(v, k):
    idx = jnp.full((_LD,), k, jnp.int32)
    return v.at[idx].get(mode="promise_in_bounds")


@functools.partial(
    pl.kernel,
    out_type=jax.ShapeDtypeStruct((128,), jnp.float32),
    mesh=plsc.VectorSubcoreMesh(core_axis_name="c", subcore_axis_name="s"),
    scratch_types=[
        pltpu.VMEM((_ROWS, _LD), jnp.float32),
        pltpu.VMEM((_N, _LD), jnp.float32),
        pltpu.VMEM((_N, _LD), jnp.float32),
        pltpu.VMEM((_N * _N, _LD), jnp.float32),
        pltpu.VMEM((_LD,), jnp.float32),
        pltpu.VMEM((128,), jnp.float32),
    ],
)
def _solver(w_hbm, out_hbm, w_v, h_v, e_v, ewc_v, pv_v, out_v):
    c = lax.axis_index("c")
    s = lax.axis_index("s")

    @pl.when(jnp.logical_and(c == 0, s == 0))
    def _():
        pltpu.sync_copy(w_hbm, w_v)
        zero = jnp.zeros((_LD,), jnp.float32)
        for i in range(_N):
            h_v[i] = zero
        pv_v[...] = zero

        wc = w_v[_R_WMSG + 2 * _LD]
        drows = [w_v[_R_DATA + i] for i in range(_N)]
        for x in range(_N):
            for y in range(x, _N):
                d = drows[x] - drows[y]
                s2 = _lanesum(d * d) + 1e-12
                vec = _sqrt16(s2) * wc
                ewc_v[x * _N + y] = vec
                if y != x:
                    ewc_v[y * _N + x] = vec

        lane = lax.iota(jnp.int32, 16)

        def step(_, carry):
            wenc = [w_v[_R_WENC + r] for r in range(17)]
            benc = w_v[_R_BENC]
            pvec = pv_v[...]
            hrows = [h_v[i] for i in range(_N)]
            Es = []
            for i in range(_N):
                acc = benc + pvec[i] * wenc[0]
                hr = hrows[i]
                for k in range(_LD):
                    acc = acc + hr[k] * wenc[1 + k]
                e = jnp.maximum(acc, 0.0)
                e_v[i] = e
                Es.append(e)

            wms = [w_v[_R_WMSG + r] for r in range(2 * _LD)]
            As = []
            Bs = []
            for i in range(_N):
                er = Es[i]
                ek0 = er[0]
                sa = ek0 * wms[0]
                sb = ek0 * wms[_LD]
                for k in range(1, _LD):
                    ek = er[k]
                    sa = sa + ek * wms[k]
                    sb = sb + ek * wms[_LD + k]
                As.append(sa)
                Bs.append(sb)

            bm = w_v[_R_BMSG]
            aggs = []
            for y in range(_N):
                base = Bs[y] + bm
                agg = jnp.maximum(As[0] + base + ewc_v[y], 0.0)
                for x in range(1, _N):
                    m = jnp.maximum(As[x] + base + ewc_v[x * _N + y], 0.0)
                    agg = jnp.maximum(agg, m)
                aggs.append(agg)

            bu = w_v[_R_BUPD]
            acc = [bu for _ in range(_N)]
            for base_r, rows in ((_R_WUPD, Es), (_R_WUPD + _LD, aggs),
                                 (_R_WUPD + 2 * _LD, hrows)):
                for k in range(_LD):
                    w = w_v[base_r + k]
                    for i in range(_N):
                        acc[i] = acc[i] + rows[i][k] * w
            hs = [jnp.maximum(a, 0.0) for a in acc]
            for i in range(_N):
                h_v[i] = hs[i]

            wm1 = w_v[_R_WMST]
            wm2 = w_v[_R_WMST + 1]
            best = _lanesum(Es[0] * wm1 + hs[0] * wm2)
            bidx = jnp.zeros((_LD,), jnp.int32)
            for i in range(1, _N):
                m = _lanesum(Es[i] * wm1 + hs[i] * wm2)
                gt = m > best
                best = jnp.where(gt, m, best)
                bidx = jnp.where(gt, jnp.full((_LD,), i, jnp.int32), bidx)
            pv_v[...] = jnp.where(lane == bidx, jnp.float32(1.0), pvec)
            return carry

        lax.fori_loop(0, _N - 1, step, jnp.int32(0))

        wp = [w_v[_R_WPRED + r] for r in range(4)]
        bp = _bcast(w_v[_R_SCAL], 1)
        us = []
        vs = []
        for i in range(_N):
            e = e_v[i]
            hh = h_v[i]
            us.append(_lanesum(e * wp[0] + hh * wp[1]) + bp)
            vs.append(_lanesum(e * wp[2] + hh * wp[3]))
        ob = [jnp.zeros((_LD,), jnp.float32)] * 8
        for x in range(_N):
            for y in range(_N):
                e = x * _N + y
                ob[e // _LD] = jnp.where(lane == (e % _LD), us[x] + vs[y],
                                         ob[e // _LD])
        for b in range(8):
            out_v[pl.ds(b * _LD, _LD)] = ob[b]
        pltpu.sync_copy(out_v, out_hbm)


def kernel(data, W_enc, b_enc, W_msg, b_msg, W_upd, b_upd, W_mst, b_mst,
           W_pred, b_pred):
    scal = jnp.zeros((_LD,), jnp.float32)
    scal = scal.at[0].set(b_mst[0]).at[1].set(b_pred[0])
    packed = jnp.concatenate(
        [
            W_enc.astype(jnp.float32),
            W_msg.astype(jnp.float32),
            W_upd.astype(jnp.float32),
            b_enc.astype(jnp.float32)[None],
            b_msg.astype(jnp.float32)[None],
            b_upd.astype(jnp.float32)[None],
            W_mst.astype(jnp.float32).reshape(2, _LD),
            W_pred.astype(jnp.float32).reshape(4, _LD),
            scal[None],
            data.astype(jnp.float32),
            jnp.zeros((1, _LD), jnp.float32),
        ],
        axis=0,
    )
    return _solver(packed)[: _N * _N]


# speedup vs baseline: 4.8001x; 1.0268x over previous
import functools

import jax
import jax.numpy as jnp
from jax import lax
from jax.experimental import pallas as pl
from jax.experimental.pallas import tpu as pltpu
from jax.experimental.pallas import tpu_sc as plsc

_N = 11
_LD = 16

_R_WENC = 0
_R_WMSG = 17
_R_WUPD = 50
_R_BENC = 98
_R_BMSG = 99
_R_BUPD = 100
_R_WMST = 101
_R_WPRED = 103
_R_SCAL = 107
_R_DATA = 108
_ROWS = 120


def _sqrt16(a):
    i = lax.bitcast_convert_type(a, jnp.int32)
    i = 0x5F3759DF - lax.shift_right_logical(i, 1)
    y = lax.bitcast_convert_type(i, jnp.float32)
    for _ in range(3):
        y = y * (1.5 - 0.5 * a * y * y)
    return a * y


def _lanesum(v):
    lane = lax.iota(jnp.int32, _LD)
    for sh in (1, 2, 4, 8):
        idx = jnp.bitwise_xor(lane, jnp.int32(sh))
        v = v + v.at[idx].get(mode="promise_in_bounds")
    return v


def _bcast(v, k):
    idx = jnp.full((_LD,), k, jnp.int32)
    return v.at[idx].get(mode="promise_in_bounds")


@functools.partial(
    pl.kernel,
    out_type=jax.ShapeDtypeStruct((128,), jnp.float32),
    mesh=plsc.VectorSubcoreMesh(core_axis_name="c", subcore_axis_name="s",
                                num_cores=1),
    scratch_types=[
        pltpu.VMEM((_ROWS, _LD), jnp.float32),
        pltpu.VMEM((_N, _LD), jnp.float32),
        pltpu.VMEM((_N, _LD), jnp.float32),
        pltpu.VMEM((_N * _N, _LD), jnp.float32),
        pltpu.VMEM((_LD,), jnp.float32),
        pltpu.VMEM((128,), jnp.float32),
    ],
)
def _solver(w_hbm, out_hbm, w_v, h_v, e_v, ewc_v, pv_v, out_v):
    c = lax.axis_index("c")
    s = lax.axis_index("s")

    @pl.when(jnp.logical_and(c == 0, s == 0))
    def _():
        pltpu.sync_copy(w_hbm, w_v)
        zero = jnp.zeros((_LD,), jnp.float32)
        for i in range(_N):
            h_v[i] = zero
        pv_v[...] = zero

        wc = w_v[_R_WMSG + 2 * _LD]
        drows = [w_v[_R_DATA + i] for i in range(_N)]
        for x in range(_N):
            for y in range(x, _N):
                d = drows[x] - drows[y]
                s2 = _lanesum(d * d) + 1e-12
                vec = _sqrt16(s2) * wc
                ewc_v[x * _N + y] = vec
                if y != x:
                    ewc_v[y * _N + x] = vec

        lane = lax.iota(jnp.int32, 16)

        def step(_, carry):
            wenc = [w_v[_R_WENC + r] for r in range(17)]
            benc = w_v[_R_BENC]
            pvec = pv_v[...]
            hrows = [h_v[i] for i in range(_N)]
            Es = []
            for i in range(_N):
                acc = benc + pvec[i] * wenc[0]
                hr = hrows[i]
                for k in range(_LD):
                    acc = acc + hr[k] * wenc[1 + k]
                e = jnp.maximum(acc, 0.0)
                e_v[i] = e
                Es.append(e)

            wms = [w_v[_R_WMSG + r] for r in range(2 * _LD)]
            As = []
            Bs = []
            for i in range(_N):
                er = Es[i]
                ek0 = er[0]
                sa = ek0 * wms[0]
                sb = ek0 * wms[_LD]
                for k in range(1, _LD):
                    ek = er[k]
                    sa = sa + ek * wms[k]
                    sb = sb + ek * wms[_LD + k]
                As.append(sa)
                Bs.append(sb)

            bm = w_v[_R_BMSG]
            aggs = []
            for y in range(_N):
                base = Bs[y] + bm
                agg = jnp.maximum(As[0] + base + ewc_v[y], 0.0)
                for x in range(1, _N):
                    m = jnp.maximum(As[x] + base + ewc_v[x * _N + y], 0.0)
                    agg = jnp.maximum(agg, m)
                aggs.append(agg)

            bu = w_v[_R_BUPD]
            acc = [bu for _ in range(_N)]
            for base_r, rows in ((_R_WUPD, Es), (_R_WUPD + _LD, aggs),
                                 (_R_WUPD + 2 * _LD, hrows)):
                for k in range(_LD):
                    w = w_v[base_r + k]
                    for i in range(_N):
                        acc[i] = acc[i] + rows[i][k] * w
            hs = [jnp.maximum(a, 0.0) for a in acc]
            for i in range(_N):
                h_v[i] = hs[i]

            wm1 = w_v[_R_WMST]
            wm2 = w_v[_R_WMST + 1]
            best = _lanesum(Es[0] * wm1 + hs[0] * wm2)
            bidx = jnp.zeros((_LD,), jnp.int32)
            for i in range(1, _N):
                m = _lanesum(Es[i] * wm1 + hs[i] * wm2)
                gt = m > best
                best = jnp.where(gt, m, best)
                bidx = jnp.where(gt, jnp.full((_LD,), i, jnp.int32), bidx)
            pv_v[...] = jnp.where(lane == bidx, jnp.float32(1.0), pvec)
            return carry

        lax.fori_loop(0, _N - 1, step, jnp.int32(0))

        wp = [w_v[_R_WPRED + r] for r in range(4)]
        bp = _bcast(w_v[_R_SCAL], 1)
        us = []
        vs = []
        for i in range(_N):
            e = e_v[i]
            hh = h_v[i]
            us.append(_lanesum(e * wp[0] + hh * wp[1]) + bp)
            vs.append(_lanesum(e * wp[2] + hh * wp[3]))
        ob = [jnp.zeros((_LD,), jnp.float32)] * 8
        for x in range(_N):
            for y in range(_N):
                e = x * _N + y
                ob[e // _LD] = jnp.where(lane == (e % _LD), us[x] + vs[y],
                                         ob[e // _LD])
        for b in range(8):
            out_v[pl.ds(b * _LD, _LD)] = ob[b]
        pltpu.sync_copy(out_v, out_hbm)


def kernel(data, W_enc, b_enc, W_msg, b_msg, W_upd, b_upd, W_mst, b_mst,
           W_pred, b_pred):
    scal = jnp.zeros((_LD,), jnp.float32)
    scal = scal.at[0].set(b_mst[0]).at[1].set(b_pred[0])
    packed = jnp.concatenate(
        [
            W_enc.astype(jnp.float32),
            W_msg.astype(jnp.float32),
            W_upd.astype(jnp.float32),
            b_enc.astype(jnp.float32)[None],
            b_msg.astype(jnp.float32)[None],
            b_upd.astype(jnp.float32)[None],
            W_mst.astype(jnp.float32).reshape(2, _LD),
            W_pred.astype(jnp.float32).reshape(4, _LD),
            scal[None],
            data.astype(jnp.float32),
            jnp.zeros((1, _LD), jnp.float32),
        ],
        axis=0,
    )
    return _solver(packed)[: _N * _N]


# speedup vs baseline: 6.0728x; 1.2652x over previous
import functools

import jax
import jax.numpy as jnp
from jax import lax
from jax.experimental import pallas as pl
from jax.experimental.pallas import tpu as pltpu
from jax.experimental.pallas import tpu_sc as plsc

_N = 11
_LD = 16

_R_WENC = 0
_R_WMSG = 17
_R_WUPD = 50
_R_BENC = 98
_R_BMSG = 99
_R_BUPD = 100
_R_WMST = 101
_R_WPRED = 103
_R_DATA = 107
_ROWS = 120


def _sqrt16(a):
    i = lax.bitcast_convert_type(a, jnp.int32)
    i = 0x5F3759DF - lax.shift_right_logical(i, 1)
    y = lax.bitcast_convert_type(i, jnp.float32)
    for _ in range(3):
        y = y * (1.5 - 0.5 * a * y * y)
    return a * y


def _lanesum(v):
    lane = lax.iota(jnp.int32, _LD)
    for sh in (1, 2, 4, 8):
        idx = jnp.bitwise_xor(lane, jnp.int32(sh))
        v = v + v.at[idx].get(mode="promise_in_bounds")
    return v


@functools.partial(
    pl.kernel,
    out_type=jax.ShapeDtypeStruct((_N * _N,), jnp.float32),
    mesh=plsc.VectorSubcoreMesh(core_axis_name="c", subcore_axis_name="s",
                                num_cores=1),
    scratch_types=[
        pltpu.VMEM((_ROWS, _LD), jnp.float32),
        pltpu.VMEM((_N, _LD), jnp.float32),
        pltpu.VMEM((_N, _LD), jnp.float32),
        pltpu.VMEM((_N * _N, _LD), jnp.float32),
        pltpu.VMEM((_LD,), jnp.float32),
        pltpu.VMEM((128,), jnp.float32),
        pltpu.VMEM((_LD,), jnp.float32),
        pltpu.SemaphoreType.DMA,
    ],
)
def _solver(we_h, wm_h, wu_h, be_h, bm_h, bu_h, wmst_h, wpred_h, bp_h, d_h,
            out_h, w_v, h_v, e_v, ewc_v, pv_v, out_v, bp_v, sem):
    c = lax.axis_index("c")
    s = lax.axis_index("s")

    @pl.when(jnp.logical_and(c == 0, s == 0))
    def _():
        copies = [
            pltpu.async_copy(we_h, w_v.at[pl.ds(_R_WENC, 17)], sem),
            pltpu.async_copy(wm_h, w_v.at[pl.ds(_R_WMSG, 33)], sem),
            pltpu.async_copy(wu_h, w_v.at[pl.ds(_R_WUPD, 48)], sem),
            pltpu.async_copy(be_h, w_v.at[pl.ds(_R_BENC, 1)], sem),
            pltpu.async_copy(bm_h, w_v.at[pl.ds(_R_BMSG, 1)], sem),
            pltpu.async_copy(bu_h, w_v.at[pl.ds(_R_BUPD, 1)], sem),
            pltpu.async_copy(wmst_h, w_v.at[pl.ds(_R_WMST, 2)], sem),
            pltpu.async_copy(wpred_h, w_v.at[pl.ds(_R_WPRED, 4)], sem),
            pltpu.async_copy(bp_h, bp_v.at[pl.ds(0, 1)], sem),
            pltpu.async_copy(d_h, w_v.at[pl.ds(_R_DATA, _N)], sem),
        ]
        zero = jnp.zeros((_LD,), jnp.float32)
        for i in range(_N):
            h_v[i] = zero
        pv_v[...] = zero
        for cp in copies:
            cp.wait()

        wc = w_v[_R_WMSG + 2 * _LD]
        drows = [w_v[_R_DATA + i] for i in range(_N)]
        for x in range(_N):
            for y in range(x, _N):
                d = drows[x] - drows[y]
                s2 = _lanesum(d * d) + 1e-12
                vec = _sqrt16(s2) * wc
                ewc_v[x * _N + y] = vec
                if y != x:
                    ewc_v[y * _N + x] = vec

        lane = lax.iota(jnp.int32, 16)

        def step(_, carry):
            wenc = [w_v[_R_WENC + r] for r in range(17)]
            benc = w_v[_R_BENC]
            pvec = pv_v[...]
            hrows = [h_v[i] for i in range(_N)]
            Es = []
            for i in range(_N):
                acc = benc + pvec[i] * wenc[0]
                hr = hrows[i]
                for k in range(_LD):
                    acc = acc + hr[k] * wenc[1 + k]
                e = jnp.maximum(acc, 0.0)
                e_v[i] = e
                Es.append(e)

            wms = [w_v[_R_WMSG + r] for r in range(2 * _LD)]
            bm = w_v[_R_BMSG]
            As = []
            Bs = []
            for i in range(_N):
                er = Es[i]
                ek0 = er[0]
                sa = ek0 * wms[0]
                sb = bm + ek0 * wms[_LD]
                for k in range(1, _LD):
                    ek = er[k]
                    sa = sa + ek * wms[k]
                    sb = sb + ek * wms[_LD + k]
                As.append(sa)
                Bs.append(sb)

            aggs = []
            for y in range(_N):
                base = Bs[y]
                agg = As[0] + base + ewc_v[y]
                for x in range(1, _N):
                    agg = jnp.maximum(agg, As[x] + base + ewc_v[x * _N + y])
                aggs.append(jnp.maximum(agg, 0.0))

            bu = w_v[_R_BUPD]
            acc = [bu for _ in range(_N)]
            for base_r, rows in ((_R_WUPD, Es), (_R_WUPD + _LD, aggs),
                                 (_R_WUPD + 2 * _LD, hrows)):
                for k in range(_LD):
                    w = w_v[base_r + k]
                    for i in range(_N):
                        acc[i] = acc[i] + rows[i][k] * w
            hs = [jnp.maximum(a, 0.0) for a in acc]
            for i in range(_N):
                h_v[i] = hs[i]

            wm1 = w_v[_R_WMST]
            wm2 = w_v[_R_WMST + 1]
            best = _lanesum(Es[0] * wm1 + hs[0] * wm2)
            bidx = jnp.zeros((_LD,), jnp.int32)
            for i in range(1, _N):
                m = _lanesum(Es[i] * wm1 + hs[i] * wm2)
                gt = m > best
                best = jnp.where(gt, m, best)
                bidx = jnp.where(gt, jnp.full((_LD,), i, jnp.int32), bidx)
            pv_v[...] = jnp.where(lane == bidx, jnp.float32(1.0), pvec)
            return carry

        lax.fori_loop(0, _N - 1, step, jnp.int32(0))

        wp = [w_v[_R_WPRED + r] for r in range(4)]
        bp = bp_v[...][0]
        us = []
        vs = []
        for i in range(_N):
            e = e_v[i]
            hh = h_v[i]
            us.append(_lanesum(e * wp[0] + hh * wp[1]) + bp)
            vs.append(_lanesum(e * wp[2] + hh * wp[3]))
        ob = [jnp.zeros((_LD,), jnp.float32)] * 8
        for x in range(_N):
            for y in range(_N):
                e = x * _N + y
                ob[e // _LD] = jnp.where(lane == (e % _LD), us[x] + vs[y],
                                         ob[e // _LD])
        for b in range(8):
            out_v[pl.ds(b * _LD, _LD)] = ob[b]
        pltpu.sync_copy(out_v.at[pl.ds(0, _N * _N)], out_h)


def kernel(data, W_enc, b_enc, W_msg, b_msg, W_upd, b_upd, W_mst, b_mst,
           W_pred, b_pred):
    f = jnp.float32
    return _solver(
        W_enc.astype(f), W_msg.astype(f), W_upd.astype(f),
        b_enc.astype(f).reshape(1, _LD), b_msg.astype(f).reshape(1, _LD),
        b_upd.astype(f).reshape(1, _LD), W_mst.astype(f).reshape(2, _LD),
        W_pred.astype(f).reshape(4, _LD), b_pred.astype(f), data.astype(f),
    )
